# baseline (speedup 1.0000x reference)
import functools

import jax
import jax.numpy as jnp
from jax import lax
from jax.experimental import pallas as pl
from jax.experimental.pallas import tpu as pltpu
from jax.experimental.pallas import tpu_sc as plsc

_B, _M, _FEAT, _ENC = 16, 2048, 3, 256
_H1, _H2 = 128, 256
_N = _B * _M
_PHASES = 4
_CHUNK = 256
_NCH = _M // _CHUNK
_ROWLEN = _M * _FEAT


_XPADW = 16

_GDN = lax.GatherDimensionNumbers(offset_dims=(), collapsed_slice_dims=(0,),
                                  start_index_map=(0,))


def _take16(v, idx):
    return lax.gather(v, idx[:, None], dimension_numbers=_GDN,
                      slice_sizes=(1,),
                      mode=lax.GatherScatterMode.PROMISE_IN_BOUNDS)


def _prefix16(v, lanes):
    for d in (1, 2, 4, 8):
        shifted = _take16(v, jnp.maximum(lanes - d, 0))
        ge = jnp.minimum(jnp.maximum(lanes - (d - 1), 0), 1)
        v = v + shifted * ge
    return v


def _sc_body(xp_hbm, mask_hbm, xc_hbm, cnt_hbm, mvec, idxg, gat, cntv, sem):
    wid = lax.axis_index("s") * 2 + lax.axis_index("c")

    @pl.when(wid < _B)
    def _work():
        base_tok = wid * _M
        pltpu.sync_copy(mask_hbm.at[pl.ds(base_tok, _M)], mvec)
        fill = jnp.full((16,), base_tok, jnp.int32)

        def init(j, _):
            idxg[pl.ds(j * 16, 16)] = fill
            return 0

        lax.fori_loop(0, _M // 16, init, 0)

        lanes = lax.iota(jnp.int32, 16)

        def step(j, carry):
            cnt, packed = carry
            mv = mvec[pl.ds(j * 16, 16)]
            pos = _prefix16(mv, lanes)
            dst = pos - 1
            gidv = base_tok + j * 16 + lanes
            for l in range(16):
                eq = 1 - jnp.minimum(jnp.abs(lanes - dst[l]), 1)
                sel = eq * mv[l]
                packed = packed + (gidv[l] - packed) * sel
            idxg[pl.ds(cnt, 16)] = packed
            return cnt + pos[15], packed

        total, _ = lax.fori_loop(
            0, _M // 16, step,
            (jnp.int32(0), jnp.full((16,), base_tok, jnp.int32)))
        cntv[...] = lanes * 0 + total
        pltpu.async_copy(xp_hbm.at[idxg], gat, sem).wait()
        pltpu.sync_copy(gat, xc_hbm.at[wid])
        pltpu.sync_copy(cntv, cnt_hbm.at[wid])


def _sc_compact(x_pad, mask_i32):
    mesh = plsc.VectorSubcoreMesh(core_axis_name="c", subcore_axis_name="s")
    kern = functools.partial(
        pl.kernel,
        mesh=mesh,
        out_type=(
            jax.ShapeDtypeStruct((_B, _M, _XPADW), jnp.float32),
            jax.ShapeDtypeStruct((_B, 16), jnp.int32),
        ),
        scratch_types=[
            pltpu.VMEM((_M,), jnp.int32),
            pltpu.VMEM((_M,), jnp.int32),
            pltpu.VMEM((_M, _XPADW), jnp.float32),
            pltpu.VMEM((16,), jnp.int32),
            pltpu.SemaphoreType.DMA,
        ],
        compiler_params=pltpu.CompilerParams(use_tc_tiling_on_sc=False),
    )(_sc_body)
    return kern(x_pad, mask_i32)


def _tc_body(cnt_ref, x_ref, W1_ref, b1_ref, g1_ref, be1_ref, W2_ref,
             b2_ref, W3_ref, b3_ref, g2_ref, be2_ref, W4_ref, b4_ref,
             out_ref,
             h1p, hm, h2p, pooled, pp, cnt_v, sum1, sq1, scale1,
             shift1, sum2, sq2, scale2, shift2):
    s = pl.program_id(0)
    i = lax.rem(s, _B)
    phase = lax.div(s, _B)
    ci = cnt_ref[i, 0]
    nch = jnp.maximum(lax.div(ci + (_CHUNK - 1), _CHUNK), 1)
    rowid = lax.broadcasted_iota(jnp.int32, (_CHUNK, 1), 0)

    @pl.when(s == 0)
    def _init():
        cnt_v[...] = jnp.zeros_like(cnt_v)
        sum1[...] = jnp.zeros_like(sum1)
        sq1[...] = jnp.zeros_like(sq1)
        sum2[...] = jnp.zeros_like(sum2)
        sq2[...] = jnp.zeros_like(sq2)

    @pl.when(phase == 0)
    def _p1():
        cnt_v[...] += ci.astype(jnp.float32)

        def chunk(k, _):
            r = pl.ds(i * _M + k * _CHUNK, _CHUNK)
            xb = x_ref[pl.ds(k * _CHUNK, _CHUNK), :]
            valid = (rowid + k * _CHUNK) < ci
            h = jnp.dot(xb, W1_ref[...], preferred_element_type=jnp.float32)
            h = h + b1_ref[...]
            h1p[r, :] = h.astype(jnp.bfloat16)
            hmask = jnp.where(valid, h, 0.0)
            sum1[...] += jnp.sum(hmask, axis=0, keepdims=True)
            sq1[...] += jnp.sum(hmask * h, axis=0, keepdims=True)
            return 0

        lax.fori_loop(0, nch, chunk, 0)

    @pl.when(jnp.logical_and(phase == 1, i == 0))
    def _fin1():
        inv = 1.0 / cnt_v[:, :1]
        mean = sum1[...] * inv
        var = sq1[...] * inv - mean * mean
        sc = g1_ref[...] * jax.lax.rsqrt(var + 1e-5)
        scale1[...] = sc
        shift1[...] = be1_ref[...] - mean * sc

    @pl.when(phase == 1)
    def _p2():
        pooled[pl.ds(i, 1), :] = jnp.broadcast_to(
            jnp.where(ci < _M, 0.0, -jnp.inf), (1, _H2))

        def chunk(k, _):
            r = pl.ds(i * _M + k * _CHUNK, _CHUNK)
            valid = (rowid + k * _CHUNK) < ci
            hp = h1p[r, :].astype(jnp.float32)
            hn = jnp.maximum(hp * scale1[...] + shift1[...], 0.0)
            hv = jnp.dot(hn.astype(jnp.bfloat16),
                         W2_ref[...].astype(jnp.bfloat16),
                         preferred_element_type=jnp.float32)
            hmv = jnp.where(valid, hv + b2_ref[...], 0.0)
            hm[r, :] = hmv.astype(jnp.bfloat16)
            pm = jnp.max(hmv, axis=0, keepdims=True)
            pooled[pl.ds(i, 1), :] = jnp.maximum(pooled[pl.ds(i, 1), :], pm)
            return 0

        lax.fori_loop(0, nch, chunk, 0)

    @pl.when(jnp.logical_and(phase == 2, i == 0))
    def _pp():
        pp[...] = jnp.dot(pooled[...], W3_ref[_H2:, :],
                          preferred_element_type=jnp.float32) + b3_ref[...]

    @pl.when(phase == 2)
    def _p3():
        def chunk(k, _):
            r = pl.ds(i * _M + k * _CHUNK, _CHUNK)
            valid = (rowid + k * _CHUNK) < ci
            hv = hm[r, :]
            h2 = jnp.dot(hv, W3_ref[:_H2, :].astype(jnp.bfloat16),
                         preferred_element_type=jnp.float32)
            h2 = h2 + pp[pl.ds(i, 1), :]
            h2p[r, :] = h2.astype(jnp.bfloat16)
            h2m = jnp.where(valid, h2, 0.0)
            sum2[...] += jnp.sum(h2m, axis=0, keepdims=True)
            sq2[...] += jnp.sum(h2m * h2, axis=0, keepdims=True)
            return 0

        lax.fori_loop(0, nch, chunk, 0)

    @pl.when(jnp.logical_and(phase == 3, i == 0))
    def _fin2():
        inv = 1.0 / cnt_v[:, :1]
        mean = sum2[...] * inv
        var = sq2[...] * inv - mean * mean
        sc = g2_ref[...] * jax.lax.rsqrt(var + 1e-5)
        scale2[...] = sc
        shift2[...] = be2_ref[...] - mean * sc

    @pl.when(phase == 3)
    def _p4():
        out_ref[pl.ds(i, 1), :] = jnp.broadcast_to(
            jnp.where(ci < _M, 0.0, -jnp.inf), (1, _ENC))

        def chunk(k, _):
            r = pl.ds(i * _M + k * _CHUNK, _CHUNK)
            valid = (rowid + k * _CHUNK) < ci
            h2 = h2p[r, :].astype(jnp.float32)
            h2n = jnp.maximum(h2 * scale2[...] + shift2[...], 0.0)
            o = jnp.dot(h2n.astype(jnp.bfloat16),
                        W4_ref[...].astype(jnp.bfloat16),
                        preferred_element_type=jnp.float32)
            om = jnp.where(valid, o + b4_ref[...], 0.0)
            mx = jnp.max(om, axis=0, keepdims=True)
            out_ref[pl.ds(i, 1), :] = jnp.maximum(out_ref[pl.ds(i, 1), :], mx)
            return 0

        lax.fori_loop(0, nch, chunk, 0)


def kernel(x, mask, W1, b1, g1, be1, W2, b2, W3, b3, g2, be2, W4, b4):
    x_pad = jnp.pad(x.reshape(_N, _FEAT), ((0, 0), (0, _XPADW - _FEAT)))
    W1p = jnp.pad(W1, ((0, _XPADW - _FEAT), (0, 0)))
    mask_i32 = mask.reshape(_N).astype(jnp.int32)
    xc3, cnts = _sc_compact(x_pad, mask_i32)
    xc = xc3.reshape(_N, _XPADW)

    row_spec = pl.BlockSpec((_M, _XPADW),
                            lambda s, c: (jnp.minimum(s, _B - 1), 0))

    def full(a):
        return pl.BlockSpec(a.shape, lambda s, c: (0,) * a.ndim)

    b1r, g1r, be1r = b1.reshape(1, _H1), g1.reshape(1, _H1), be1.reshape(1, _H1)
    b2r = b2.reshape(1, _H2)
    b3r, g2r, be2r = b3.reshape(1, _H2), g2.reshape(1, _H2), be2.reshape(1, _H2)
    b4r = b4.reshape(1, _ENC)
    ops = (xc, W1p, b1r, g1r, be1r, W2, b2r, W3, b3r, g2r, be2r, W4, b4r)
    in_specs = [row_spec] + [full(a) for a in ops[1:]]

    grid_spec = pltpu.PrefetchScalarGridSpec(
        num_scalar_prefetch=1,
        grid=(_PHASES * _B,),
        in_specs=in_specs,
        out_specs=pl.BlockSpec((_B, _ENC), lambda s, c: (0, 0)),
        scratch_shapes=[
            pltpu.VMEM((_N, _H1), jnp.bfloat16),
            pltpu.VMEM((_N, _H2), jnp.bfloat16),
            pltpu.VMEM((_N, _H2), jnp.bfloat16),
            pltpu.VMEM((_B, _H2), jnp.float32),
            pltpu.VMEM((_B, _H2), jnp.float32),
            pltpu.VMEM((1, _H1), jnp.float32),
            pltpu.VMEM((1, _H1), jnp.float32),
            pltpu.VMEM((1, _H1), jnp.float32),
            pltpu.VMEM((1, _H1), jnp.float32),
            pltpu.VMEM((1, _H1), jnp.float32),
            pltpu.VMEM((1, _H2), jnp.float32),
            pltpu.VMEM((1, _H2), jnp.float32),
            pltpu.VMEM((1, _H2), jnp.float32),
            pltpu.VMEM((1, _H2), jnp.float32),
        ],
    )
    out = pl.pallas_call(
        _tc_body,
        grid_spec=grid_spec,
        out_shape=jax.ShapeDtypeStruct((_B, _ENC), jnp.float32),
        compiler_params=pltpu.CompilerParams(
            vmem_limit_bytes=100 * 1024 * 1024,
        ),
    )(cnts, *ops)
    return out


# speedup vs baseline: 2.4118x; 2.4118x over previous
import jax
import jax.numpy as jnp
from jax.experimental import pallas as pl
from jax.experimental.pallas import tpu as pltpu

_B, _M, _FEAT, _ENC = 16, 2048, 3, 256
_H1, _H2 = 128, 256
_N = _B * _M
_PHASES = 4
_SPS = 2
_NSTEP = _B // _SPS
_BLK = _SPS * _M


def _body(x_ref, mc_ref, W1_ref, b1_ref, g1_ref, be1_ref, W2_ref,
          b2_ref, W3_ref, b3_ref, g2_ref, be2_ref, W4_ref, b4_ref,
          out_ref,
          h1p, hm, h2p, mstash, pooled, pp, cnt_v, sum1, sq1, scale1,
          shift1, sum2, sq2, scale2, shift2):
    s = pl.program_id(0)
    i = jax.lax.rem(s, _NSTEP)
    phase = jax.lax.div(s, _NSTEP)

    def rows(h):
        return pl.ds((i * _SPS + h) * _M, _M)

    def segs(h):
        return pl.ds(i * _SPS + h, 1)

    @pl.when(s == 0)
    def _init():
        cnt_v[...] = jnp.zeros_like(cnt_v)
        sum1[...] = jnp.zeros_like(sum1)
        sq1[...] = jnp.zeros_like(sq1)
        sum2[...] = jnp.zeros_like(sum2)
        sq2[...] = jnp.zeros_like(sq2)

    @pl.when(phase == 0)
    def _p1():
        xa = x_ref[...]
        ma = mc_ref[...].astype(jnp.float32)
        for h in range(_SPS):
            xb = xa[h * _M:(h + 1) * _M, :]
            m = ma[h * _M:(h + 1) * _M, :]
            mstash[rows(h), :] = m.astype(jnp.bfloat16)
            hh = jnp.dot(xb, W1_ref[...], preferred_element_type=jnp.float32)
            hh = hh + b1_ref[...]
            h1p[rows(h), :] = hh.astype(jnp.bfloat16)
            hmask = hh * m
            sum1[...] += jnp.sum(hmask, axis=0, keepdims=True)
            sq1[...] += jnp.sum(hmask * hh, axis=0, keepdims=True)
            cnt_v[...] += jnp.sum(m)

    @pl.when(jnp.logical_and(phase == 1, i == 0))
    def _fin1():
        inv = 1.0 / cnt_v[:, :1]
        mean = sum1[...] * inv
        var = sq1[...] * inv - mean * mean
        sc = g1_ref[...] * jax.lax.rsqrt(var + 1e-5)
        scale1[...] = sc
        shift1[...] = be1_ref[...] - mean * sc

    @pl.when(phase == 1)
    def _p2():
        for h in range(_SPS):
            hp = h1p[rows(h), :].astype(jnp.float32)
            hn = jnp.maximum(hp * scale1[...] + shift1[...], 0.0)
            hv = jnp.dot(hn.astype(jnp.bfloat16),
                         W2_ref[...].astype(jnp.bfloat16),
                         preferred_element_type=jnp.float32)
            hv = hv + b2_ref[...]
            hmv = hv * mstash[rows(h), :].astype(jnp.float32)
            hm[rows(h), :] = hmv.astype(jnp.bfloat16)
            pooled[segs(h), :] = jnp.max(hmv, axis=0, keepdims=True)

    @pl.when(jnp.logical_and(phase == 2, i == 0))
    def _pp():
        pp[...] = jnp.dot(pooled[...], W3_ref[_H2:, :],
                          preferred_element_type=jnp.float32) + b3_ref[...]

    @pl.when(phase == 2)
    def _p3():
        for h in range(_SPS):
            hv = hm[rows(h), :]
            h2 = jnp.dot(hv, W3_ref[:_H2, :].astype(jnp.bfloat16),
                         preferred_element_type=jnp.float32)
            h2 = h2 + pp[segs(h), :]
            h2p[rows(h), :] = h2.astype(jnp.bfloat16)
            m = mstash[rows(h), :].astype(jnp.float32)
            h2m = h2 * m
            sum2[...] += jnp.sum(h2m, axis=0, keepdims=True)
            sq2[...] += jnp.sum(h2m * h2, axis=0, keepdims=True)

    @pl.when(jnp.logical_and(phase == 3, i == 0))
    def _fin2():
        inv = 1.0 / cnt_v[:, :1]
        mean = sum2[...] * inv
        var = sq2[...] * inv - mean * mean
        sc = g2_ref[...] * jax.lax.rsqrt(var + 1e-5)
        scale2[...] = sc
        shift2[...] = be2_ref[...] - mean * sc

    @pl.when(phase == 3)
    def _p4():
        for h in range(_SPS):
            h2 = h2p[rows(h), :].astype(jnp.float32)
            h2n = jnp.maximum(h2 * scale2[...] + shift2[...], 0.0)
            o = jnp.dot(h2n.astype(jnp.bfloat16),
                        W4_ref[...].astype(jnp.bfloat16),
                        preferred_element_type=jnp.float32)
            o = o + b4_ref[...]
            om = o * mstash[rows(h), :].astype(jnp.float32)
            out_ref[segs(h), :] = jnp.max(om, axis=0, keepdims=True)


def kernel(x, mask, W1, b1, g1, be1, W2, b2, W3, b3, g2, be2, W4, b4):
    x2 = x.reshape(_N, _FEAT)
    mcol = mask.reshape(_N, 1)

    def frozen_row(s):
        return (jnp.minimum(s, _NSTEP - 1), 0)

    row_spec = pl.BlockSpec((_BLK, _FEAT), frozen_row)
    mc_spec = pl.BlockSpec((_BLK, 1), frozen_row)

    def full(a):
        return pl.BlockSpec(a.shape, lambda s: (0,) * a.ndim)

    b1r, g1r, be1r = b1.reshape(1, _H1), g1.reshape(1, _H1), be1.reshape(1, _H1)
    b2r = b2.reshape(1, _H2)
    b3r, g2r, be2r = b3.reshape(1, _H2), g2.reshape(1, _H2), be2.reshape(1, _H2)
    b4r = b4.reshape(1, _ENC)
    ops = (x2, mcol, W1, b1r, g1r, be1r, W2, b2r, W3, b3r, g2r, be2r, W4, b4r)
    in_specs = [row_spec, mc_spec] + [full(a) for a in ops[2:]]

    out = pl.pallas_call(
        _body,
        grid=(_PHASES * _NSTEP,),
        in_specs=in_specs,
        out_specs=pl.BlockSpec((_B, _ENC), lambda s: (0, 0)),
        out_shape=jax.ShapeDtypeStruct((_B, _ENC), jnp.float32),
        scratch_shapes=[
            pltpu.VMEM((_N, _H1), jnp.bfloat16),
            pltpu.VMEM((_N, _H2), jnp.bfloat16),
            pltpu.VMEM((_N, _H2), jnp.bfloat16),
            pltpu.VMEM((_N, 1), jnp.bfloat16),
            pltpu.VMEM((_B, _H2), jnp.float32),
            pltpu.VMEM((_B, _H2), jnp.float32),
            pltpu.VMEM((1, _H1), jnp.float32),
            pltpu.VMEM((1, _H1), jnp.float32),
            pltpu.VMEM((1, _H1), jnp.float32),
            pltpu.VMEM((1, _H1), jnp.float32),
            pltpu.VMEM((1, _H1), jnp.float32),
            pltpu.VMEM((1, _H2), jnp.float32),
            pltpu.VMEM((1, _H2), jnp.float32),
            pltpu.VMEM((1, _H2), jnp.float32),
            pltpu.VMEM((1, _H2), jnp.float32),
        ],
        compiler_params=pltpu.CompilerParams(
            vmem_limit_bytes=100 * 1024 * 1024,
        ),
    )(*ops)
    return out


# speedup vs baseline: 2.5407x; 1.0534x over previous
import jax
import jax.numpy as jnp
from jax.experimental import pallas as pl
from jax.experimental.pallas import tpu as pltpu

_B, _M, _FEAT, _ENC = 16, 2048, 3, 256
_H1, _H2 = 128, 256
_N = _B * _M
_PHASES = 4
_SPS = 4
_NSTEP = _B // _SPS
_BLK = _SPS * _M


def _body(x_ref, mc_ref, W1_ref, b1_ref, g1_ref, be1_ref, W2_ref,
          b2_ref, W3_ref, b3_ref, g2_ref, be2_ref, W4_ref, b4_ref,
          out_ref,
          hm, h2p, mstash, pooled, pp, cnt_v, sum1, sq1, scale1,
          shift1, sum2, sq2, scale2, shift2):
    s = pl.program_id(0)
    i = jax.lax.rem(s, _NSTEP)
    phase = jax.lax.div(s, _NSTEP)

    def rows(h):
        return pl.ds((i * _SPS + h) * _M, _M)

    def segs(h):
        return pl.ds(i * _SPS + h, 1)

    @pl.when(s == 0)
    def _init():
        cnt_v[...] = jnp.zeros_like(cnt_v)
        sum1[...] = jnp.zeros_like(sum1)
        sq1[...] = jnp.zeros_like(sq1)
        sum2[...] = jnp.zeros_like(sum2)
        sq2[...] = jnp.zeros_like(sq2)

    @pl.when(phase == 0)
    def _p1():
        xa = x_ref[...]
        ma = mc_ref[...].astype(jnp.float32)
        for h in range(_SPS):
            xb = xa[h * _M:(h + 1) * _M, :]
            m = ma[h * _M:(h + 1) * _M, :]
            mstash[rows(h), :] = m.astype(jnp.bfloat16)
            hh = jnp.dot(xb, W1_ref[...], preferred_element_type=jnp.float32)
            hh = hh + b1_ref[...]
            hmask = hh * m
            sum1[...] += jnp.sum(hmask, axis=0, keepdims=True)
            sq1[...] += jnp.sum(hmask * hh, axis=0, keepdims=True)
            cnt_v[...] += jnp.sum(m)

    @pl.when(jnp.logical_and(phase == 1, i == 0))
    def _fin1():
        inv = 1.0 / cnt_v[:, :1]
        mean = sum1[...] * inv
        var = sq1[...] * inv - mean * mean
        sc = g1_ref[...] * jax.lax.rsqrt(var + 1e-5)
        scale1[...] = sc
        shift1[...] = be1_ref[...] - mean * sc

    @pl.when(phase == 1)
    def _p2():
        xa = x_ref[...]
        for h in range(_SPS):
            xb = xa[h * _M:(h + 1) * _M, :]
            hp = jnp.dot(xb, W1_ref[...],
                         preferred_element_type=jnp.float32) + b1_ref[...]
            hn = jnp.maximum(hp * scale1[...] + shift1[...], 0.0)
            hv = jnp.dot(hn.astype(jnp.bfloat16),
                         W2_ref[...].astype(jnp.bfloat16),
                         preferred_element_type=jnp.float32)
            hv = hv + b2_ref[...]
            hmv = hv * mstash[rows(h), :].astype(jnp.float32)
            hm[rows(h), :] = hmv.astype(jnp.bfloat16)
            pooled[segs(h), :] = jnp.max(hmv, axis=0, keepdims=True)

    @pl.when(jnp.logical_and(phase == 2, i == 0))
    def _pp():
        pp[...] = jnp.dot(pooled[...], W3_ref[_H2:, :],
                          preferred_element_type=jnp.float32) + b3_ref[...]

    @pl.when(phase == 2)
    def _p3():
        for h in range(_SPS):
            hv = hm[rows(h), :]
            h2 = jnp.dot(hv, W3_ref[:_H2, :].astype(jnp.bfloat16),
                         preferred_element_type=jnp.float32)
            h2 = h2 + pp[segs(h), :]
            h2p[rows(h), :] = h2.astype(jnp.bfloat16)
            m = mstash[rows(h), :].astype(jnp.float32)
            h2m = h2 * m
            sum2[...] += jnp.sum(h2m, axis=0, keepdims=True)
            sq2[...] += jnp.sum(h2m * h2, axis=0, keepdims=True)

    @pl.when(jnp.logical_and(phase == 3, i == 0))
    def _fin2():
        inv = 1.0 / cnt_v[:, :1]
        mean = sum2[...] * inv
        var = sq2[...] * inv - mean * mean
        sc = g2_ref[...] * jax.lax.rsqrt(var + 1e-5)
        scale2[...] = sc
        shift2[...] = be2_ref[...] - mean * sc

    @pl.when(phase == 3)
    def _p4():
        for h in range(_SPS):
            h2 = h2p[rows(h), :].astype(jnp.float32)
            h2n = jnp.maximum(h2 * scale2[...] + shift2[...], 0.0)
            o = jnp.dot(h2n.astype(jnp.bfloat16),
                        W4_ref[...].astype(jnp.bfloat16),
                        preferred_element_type=jnp.float32)
            o = o + b4_ref[...]
            om = o * mstash[rows(h), :].astype(jnp.float32)
            out_ref[segs(h), :] = jnp.max(om, axis=0, keepdims=True)


def kernel(x, mask, W1, b1, g1, be1, W2, b2, W3, b3, g2, be2, W4, b4):
    x2 = x.reshape(_N, _FEAT)
    mcol = mask.reshape(_N, 1)

    def frozen_row(s):
        return (jnp.minimum(s, _NSTEP - 1), 0)

    def x_row(s):
        return (jnp.where(s < 2 * _NSTEP, jax.lax.rem(s, _NSTEP),
                          _NSTEP - 1), 0)

    row_spec = pl.BlockSpec((_BLK, _FEAT), x_row)
    mc_spec = pl.BlockSpec((_BLK, 1), frozen_row)

    def full(a):
        return pl.BlockSpec(a.shape, lambda s: (0,) * a.ndim)

    b1r, g1r, be1r = b1.reshape(1, _H1), g1.reshape(1, _H1), be1.reshape(1, _H1)
    b2r = b2.reshape(1, _H2)
    b3r, g2r, be2r = b3.reshape(1, _H2), g2.reshape(1, _H2), be2.reshape(1, _H2)
    b4r = b4.reshape(1, _ENC)
    ops = (x2, mcol, W1, b1r, g1r, be1r, W2, b2r, W3, b3r, g2r, be2r, W4, b4r)
    in_specs = [row_spec, mc_spec] + [full(a) for a in ops[2:]]

    out = pl.pallas_call(
        _body,
        grid=(_PHASES * _NSTEP,),
        in_specs=in_specs,
        out_specs=pl.BlockSpec((_B, _ENC), lambda s: (0, 0)),
        out_shape=jax.ShapeDtypeStruct((_B, _ENC), jnp.float32),
        scratch_shapes=[
            pltpu.VMEM((_N, _H2), jnp.bfloat16),
            pltpu.VMEM((_N, _H2), jnp.bfloat16),
            pltpu.VMEM((_N, 1), jnp.bfloat16),
            pltpu.VMEM((_B, _H2), jnp.float32),
            pltpu.VMEM((_B, _H2), jnp.float32),
            pltpu.VMEM((1, _H1), jnp.float32),
            pltpu.VMEM((1, _H1), jnp.float32),
            pltpu.VMEM((1, _H1), jnp.float32),
            pltpu.VMEM((1, _H1), jnp.float32),
            pltpu.VMEM((1, _H1), jnp.float32),
            pltpu.VMEM((1, _H2), jnp.float32),
            pltpu.VMEM((1, _H2), jnp.float32),
            pltpu.VMEM((1, _H2), jnp.float32),
            pltpu.VMEM((1, _H2), jnp.float32),
        ],
        compiler_params=pltpu.CompilerParams(
            vmem_limit_bytes=100 * 1024 * 1024,
        ),
    )(*ops)
    return out
